# baseline (speedup 1.0000x reference)
import functools

import jax
import jax.numpy as jnp
from jax import lax
from jax.experimental import pallas as pl
from jax.experimental.pallas import tpu as pltpu
from jax.experimental.pallas import tpu_sc as plsc

_B = 16384
_NUM = 13
_F = 26
_V = 100000
_D = 16
_H1 = 256
_H2 = 128
_C = 10

_NW = 32
_NTASK = _F * _D
_TPW = _NTASK // _NW
_CHB = 4096
_NCB = _B // _CHB


def _make_gather():
    mesh = plsc.VectorSubcoreMesh(core_axis_name="c", subcore_axis_name="s")

    @functools.partial(
        pl.kernel,
        mesh=mesh,
        out_type=jax.ShapeDtypeStruct((_F, _D, _B), jnp.float32),
        scratch_types=[
            pltpu.VMEM((_V,), jnp.float32),
            pltpu.VMEM((_B,), jnp.int32),
            pltpu.VMEM((_CHB,), jnp.float32),
            pltpu.VMEM((_CHB,), jnp.float32),
            pltpu.SemaphoreType.DMA,
            pltpu.SemaphoreType.DMA,
        ],
        compiler_params=pltpu.CompilerParams(needs_layout_passes=False),
    )
    def gather_k(tabT_hbm, catT_hbm, out_hbm, row_v, idx_v, val0, val1,
                 semO0, semO1):
        wid = lax.axis_index("s") * 2 + lax.axis_index("c")
        vals = (val0, val1)
        osems = (semO0, semO1)

        def do_block(cb, val_v):
            def body(i, carry):
                val_v[pl.ds(i * 16, 16)] = plsc.load_gather(
                    row_v, [idx_v[pl.ds(cb * _CHB + i * 16, 16)]])
                return carry
            lax.fori_loop(0, _CHB // 16, body, 0)

        f_prev = (wid * _TPW) // _D
        pltpu.sync_copy(catT_hbm.at[f_prev], idx_v)
        out_hs = [None, None]
        for t in range(_TPW):
            task = wid * _TPW + t
            f = task // _D
            d = task % _D

            @pl.when(f != f_prev)
            def _():
                pltpu.sync_copy(catT_hbm.at[f], idx_v)

            pltpu.sync_copy(tabT_hbm.at[f, d], row_v)
            for cb in range(_NCB):
                p = cb % 2
                if out_hs[p] is not None:
                    out_hs[p].wait()
                do_block(cb, vals[p])
                out_hs[p] = pltpu.async_copy(
                    vals[p], out_hbm.at[f, d, pl.ds(cb * _CHB, _CHB)], osems[p])
            f_prev = f
        for p in range(2):
            if out_hs[p] is not None:
                out_hs[p].wait()

    return gather_k


_gather = _make_gather()

_BN = 2048


def _mlp_body(numT_ref, embT_ref, w1nT_ref, w1eT_ref, b1_ref, w2T_ref,
              b2_ref, w3T_ref, b3_ref, outT_ref):
    h1 = jnp.dot(w1eT_ref[...].astype(jnp.bfloat16),
                 embT_ref[...].astype(jnp.bfloat16),
                 preferred_element_type=jnp.float32)
    h1 += jnp.dot(w1nT_ref[...], numT_ref[...], preferred_element_type=jnp.float32)
    h1 = jnp.maximum(h1 + b1_ref[...], 0.0)
    h2 = jnp.dot(w2T_ref[...], h1, preferred_element_type=jnp.float32)
    h2 = jnp.maximum(h2 + b2_ref[...], 0.0)
    outT_ref[...] = (
        jnp.dot(w3T_ref[...], h2, preferred_element_type=jnp.float32) + b3_ref[...]
    )


def _mlp(numT, embT, W1nT, W1eT, b1, W2T, b2, W3T, b3):
    full = lambda shape: pl.BlockSpec(shape, lambda i: (0, 0))
    return pl.pallas_call(
        _mlp_body,
        grid=(_B // _BN,),
        in_specs=[
            pl.BlockSpec((_NUM, _BN), lambda i: (0, i)),
            pl.BlockSpec((_F * _D, _BN), lambda i: (0, i)),
            full((_H1, _NUM)),
            full((_H1, _F * _D)),
            full((_H1, 1)),
            full((_H2, _H1)),
            full((_H2, 1)),
            full((_C, _H2)),
            full((_C, 1)),
        ],
        out_specs=pl.BlockSpec((_C, _BN), lambda i: (0, i)),
        out_shape=jax.ShapeDtypeStruct((_C, _B), jnp.float32),
    )(numT, embT, W1nT, W1eT, b1, W2T, b2, W3T, b3)


def kernel(num_x, cat_x, tables, W1, b1, W2, b2, W3, b3):
    tablesT = jnp.transpose(tables, (0, 2, 1))
    catT = cat_x.T
    numT = num_x.T
    embT3 = _gather(tablesT, catT)
    embT = embT3.reshape(_F * _D, _B)
    outT = _mlp(numT, embT,
                W1[:_NUM].T, W1[_NUM:].T, b1.reshape(_H1, 1),
                W2.T, b2.reshape(_H2, 1), W3.T, b3.reshape(_C, 1))
    return outT.T


# speedup vs baseline: 1.0124x; 1.0124x over previous
import functools

import jax
import jax.numpy as jnp
from jax import lax
from jax.experimental import pallas as pl
from jax.experimental.pallas import tpu as pltpu
from jax.experimental.pallas import tpu_sc as plsc

_B = 16384
_NUM = 13
_F = 26
_V = 100000
_D = 16
_H1 = 256
_H2 = 128
_C = 10

_NW = 32
_NTASK = _F * _D
_TPW = _NTASK // _NW
_CHB = 8192
_NCB = _B // _CHB


def _make_gather():
    mesh = plsc.VectorSubcoreMesh(core_axis_name="c", subcore_axis_name="s")

    @functools.partial(
        pl.kernel,
        mesh=mesh,
        out_type=jax.ShapeDtypeStruct((_F, _D, _B), jnp.float32),
        scratch_types=[
            pltpu.VMEM((_V,), jnp.float32),
            pltpu.VMEM((_B,), jnp.int32),
            pltpu.VMEM((_CHB,), jnp.float32),
        ],
        compiler_params=pltpu.CompilerParams(needs_layout_passes=False),
    )
    def gather_k(tabT_hbm, catT_hbm, out_hbm, row_v, idx_v, val_v):
        wid = lax.axis_index("s") * 2 + lax.axis_index("c")

        def do_block(cb):
            def body(i, carry):
                val_v[pl.ds(i * 16, 16)] = plsc.load_gather(
                    row_v, [idx_v[pl.ds(cb * _CHB + i * 16, 16)]])
                return carry
            lax.fori_loop(0, _CHB // 16, body, 0)

        f_prev = (wid * _TPW) // _D
        pltpu.sync_copy(catT_hbm.at[f_prev], idx_v)
        for t in range(_TPW):
            task = wid * _TPW + t
            f = task // _D
            d = task % _D

            @pl.when(f != f_prev)
            def _():
                pltpu.sync_copy(catT_hbm.at[f], idx_v)

            pltpu.sync_copy(tabT_hbm.at[f, d], row_v)
            for cb in range(_NCB):
                do_block(cb)
                pltpu.sync_copy(val_v, out_hbm.at[f, d, pl.ds(cb * _CHB, _CHB)])
            f_prev = f

    return gather_k


_gather = _make_gather()

_BN = 2048


def _mlp_body(numT_ref, embT_ref, w1nT_ref, w1eT_ref, b1_ref, w2T_ref,
              b2_ref, w3T_ref, b3_ref, outT_ref):
    h1 = jnp.dot(w1eT_ref[...].astype(jnp.bfloat16),
                 embT_ref[...].astype(jnp.bfloat16),
                 preferred_element_type=jnp.float32)
    h1 += jnp.dot(w1nT_ref[...], numT_ref[...], preferred_element_type=jnp.float32)
    h1 = jnp.maximum(h1 + b1_ref[...], 0.0)
    h2 = jnp.dot(w2T_ref[...], h1, preferred_element_type=jnp.float32)
    h2 = jnp.maximum(h2 + b2_ref[...], 0.0)
    outT_ref[...] = (
        jnp.dot(w3T_ref[...], h2, preferred_element_type=jnp.float32) + b3_ref[...]
    )


def _mlp(numT, embT, W1nT, W1eT, b1, W2T, b2, W3T, b3):
    full = lambda shape: pl.BlockSpec(shape, lambda i: (0, 0))
    return pl.pallas_call(
        _mlp_body,
        grid=(_B // _BN,),
        in_specs=[
            pl.BlockSpec((_NUM, _BN), lambda i: (0, i)),
            pl.BlockSpec((_F * _D, _BN), lambda i: (0, i)),
            full((_H1, _NUM)),
            full((_H1, _F * _D)),
            full((_H1, 1)),
            full((_H2, _H1)),
            full((_H2, 1)),
            full((_C, _H2)),
            full((_C, 1)),
        ],
        out_specs=pl.BlockSpec((_C, _BN), lambda i: (0, i)),
        out_shape=jax.ShapeDtypeStruct((_C, _B), jnp.float32),
    )(numT, embT, W1nT, W1eT, b1, W2T, b2, W3T, b3)


def kernel(num_x, cat_x, tables, W1, b1, W2, b2, W3, b3):
    tablesT = jnp.transpose(tables, (0, 2, 1))
    catT = cat_x.T
    numT = num_x.T
    embT3 = _gather(tablesT, catT)
    embT = embT3.reshape(_F * _D, _B)
    outT = _mlp(numT, embT,
                W1[:_NUM].T, W1[_NUM:].T, b1.reshape(_H1, 1),
                W2.T, b2.reshape(_H2, 1), W3.T, b3.reshape(_C, 1))
    return outT.T
